# Initial kernel scaffold; baseline (speedup 1.0000x reference)
import jax
import jax.numpy as jnp
from jax.experimental import pallas as pl


def kernel(edge_index, emb, W1, b1, W2, b2):
    raise NotImplementedError("write your pallas kernel here")


# speedup vs baseline: 11.8584x; 11.8584x over previous
import functools

import jax
import jax.numpy as jnp
from jax import lax
from jax.experimental import pallas as pl
from jax.experimental.pallas import tpu as pltpu
from jax.experimental.pallas import tpu_sc as plsc

N = 50000
E = 800000
NC = 2
NS = 16
NP = 51200
NA = 50176
RPT = NA // NS
CH = 50
TROWS = E // CH // NS
G = 100
CHD = 25
DTROWS = E // CHD // (NC * NS)
BN = 1024
NBLK = NP // BN

_mesh = functools.partial(
    plsc.VectorSubcoreMesh,
    core_axis_name="c", subcore_axis_name="s",
    num_cores=NC, num_subcores=NS,
)


@functools.partial(
    pl.kernel,
    out_type=jax.ShapeDtypeStruct((NC * NP,), jnp.float32),
    mesh=_mesh(),
    compiler_params=pltpu.CompilerParams(use_tc_tiling_on_sc=False),
    scratch_types=[
        pltpu.VMEM_SHARED((NP,), jnp.float32),
        pltpu.VMEM((DTROWS, CHD), jnp.int32),
        pltpu.VMEM((CHD,), jnp.float32),
    ],
)
def _deg_kernel(dst_hbm, zeros_hbm, ones_hbm, out_hbm, acc, dstb, onesb):
    c = lax.axis_index("c")
    s = lax.axis_index("s")
    seg = pl.ds(s * (NP // NS), NP // NS)
    pltpu.sync_copy(zeros_hbm.at[seg], acc.at[seg])
    pltpu.sync_copy(ones_hbm, onesb)
    rowbase = (c * NS + s) * DTROWS
    pltpu.sync_copy(dst_hbm.at[pl.ds(rowbase, DTROWS)], dstb)
    plsc.subcore_barrier()

    @pl.loop(0, DTROWS)
    def _(j):
        pltpu.sync_copy(onesb, acc.at[dstb.at[j]], add=True)

    plsc.subcore_barrier()
    pltpu.sync_copy(acc.at[seg], out_hbm.at[pl.ds(c * NP + s * (NP // NS), NP // NS)])


def _make_edge_pass(F):
    @functools.partial(
        pl.kernel,
        out_type=jax.ShapeDtypeStruct((2 * NP, F), jnp.float32),
        mesh=_mesh(),
        compiler_params=pltpu.CompilerParams(use_tc_tiling_on_sc=False),
        scratch_types=[
            pltpu.VMEM_SHARED((NA, F), jnp.float32),
            pltpu.VMEM((G, CH), jnp.int32),
            pltpu.VMEM((G, CH), jnp.int32),
            pltpu.VMEM((CH, F), jnp.float32),
            pltpu.SemaphoreType.DMA,
        ],
    )
    def edge_pass(y_hbm, src_hbm, dst_hbm, out_hbm, acc, srcb, dstb, rows, sem):
        c = lax.axis_index("c")
        s = lax.axis_index("s")
        base = s * RPT
        pltpu.sync_copy(y_hbm.at[pl.ds(c * NP + base, RPT)], acc.at[pl.ds(base, RPT)])
        plsc.subcore_barrier()

        @pl.loop(0, TROWS // G)
        def _(g):
            pltpu.sync_copy(
                src_hbm.at[pl.ds((c * NS + s) * TROWS + g * G, G)], srcb)
            pltpu.sync_copy(dst_hbm.at[pl.ds(s * TROWS + g * G, G)], dstb)

            @pl.loop(0, G)
            def _(j):
                pltpu.async_copy(y_hbm.at[srcb.at[j]], rows, sem).wait()
                pltpu.sync_copy(rows, acc.at[dstb.at[j]], add=True)

        plsc.subcore_barrier()
        pltpu.sync_copy(acc.at[pl.ds(base, RPT)], out_hbm.at[pl.ds(c * NP + base, RPT)])

    return edge_pass


_edge_pass_32 = _make_edge_pass(32)
_edge_pass_16 = _make_edge_pass(16)


def _dinv_block(degp_ref):
    return lax.rsqrt(degp_ref[0, 0] + degp_ref[0, 1] + 1.0)


def _y1_body(emb_ref, w_ref, degp_ref, out_ref):
    dinv = _dinv_block(degp_ref)
    xw = jnp.dot(emb_ref[...], w_ref[...], preferred_element_type=jnp.float32)
    y = xw * dinv[:, None]
    out_ref[0] = y[:, :32]
    out_ref[1] = y[:, 32:]


def _tc_y1(emb, W1, degp):
    return pl.pallas_call(
        _y1_body,
        grid=(NBLK,),
        in_specs=[
            pl.BlockSpec((BN, 64), lambda i: (i, 0)),
            pl.BlockSpec((64, 64), lambda i: (0, 0)),
            pl.BlockSpec((1, 2, BN), lambda i: (i, 0, 0)),
        ],
        out_specs=pl.BlockSpec((2, BN, 32), lambda i: (0, i, 0)),
        out_shape=jax.ShapeDtypeStruct((2, NP, 32), jnp.float32),
    )(emb, W1, degp)


def _y2_body(acc_ref, degp_ref, b1_ref, w2_ref, out_ref):
    dinv = _dinv_block(degp_ref)
    a = jnp.concatenate([acc_ref[0], acc_ref[1]], axis=1)
    h = jnp.maximum(a * dinv[:, None] + b1_ref[...], 0.0)
    xw2 = jnp.dot(h, w2_ref[...], preferred_element_type=jnp.float32)
    y2 = xw2 * dinv[:, None]
    out_ref[0] = y2[:, :16]
    out_ref[1] = y2[:, 16:]


def _tc_y2(acc1, degp, b1, W2):
    return pl.pallas_call(
        _y2_body,
        grid=(NBLK,),
        in_specs=[
            pl.BlockSpec((2, BN, 32), lambda i: (0, i, 0)),
            pl.BlockSpec((1, 2, BN), lambda i: (i, 0, 0)),
            pl.BlockSpec((1, 64), lambda i: (0, 0)),
            pl.BlockSpec((64, 32), lambda i: (0, 0)),
        ],
        out_specs=pl.BlockSpec((2, BN, 16), lambda i: (0, i, 0)),
        out_shape=jax.ShapeDtypeStruct((2, NP, 16), jnp.float32),
    )(acc1, degp, b1, W2)


def _final_body(acc_ref, degp_ref, b2_ref, out_ref):
    dinv = _dinv_block(degp_ref)
    a = jnp.concatenate([acc_ref[0], acc_ref[1]], axis=1)
    out_ref[...] = a * dinv[:, None] + b2_ref[...]


def _tc_final(acc2, degp, b2):
    return pl.pallas_call(
        _final_body,
        grid=(NBLK,),
        in_specs=[
            pl.BlockSpec((2, BN, 16), lambda i: (0, i, 0)),
            pl.BlockSpec((1, 2, BN), lambda i: (i, 0, 0)),
            pl.BlockSpec((1, 32), lambda i: (0, 0)),
        ],
        out_specs=pl.BlockSpec((BN, 32), lambda i: (i, 0)),
        out_shape=jax.ShapeDtypeStruct((NP, 32), jnp.float32),
    )(acc2, degp, b2)


@jax.jit
def kernel(edge_index, emb, W1, b1, W2, b2):
    src = edge_index[0].astype(jnp.int32)
    dst = edge_index[1].astype(jnp.int32)
    src2d = src.reshape(E // CH, CH)
    src_cat = jnp.concatenate([src2d, src2d + NP], axis=0)
    dst2d = dst.reshape(E // CH, CH)
    dstd = dst.reshape(E // CHD, CHD)

    zeros_np = jnp.zeros((NP,), jnp.float32)
    ones_c = jnp.ones((CHD,), jnp.float32)
    emb_p = jnp.pad(emb, ((0, NP - N), (0, 0)))

    degp = _deg_kernel(dstd, zeros_np, ones_c)
    degp3 = degp.reshape(2, NBLK, BN).transpose(1, 0, 2)
    y1 = _tc_y1(emb_p, W1, degp3)
    acc1 = _edge_pass_32(y1.reshape(2 * NP, 32), src_cat, dst2d)
    y2 = _tc_y2(acc1.reshape(2, NP, 32), degp3, b1.reshape(1, 64), W2)
    acc2 = _edge_pass_16(y2.reshape(2 * NP, 16), src_cat, dst2d)
    return _tc_final(acc2.reshape(2, NP, 16), degp3, b2.reshape(1, 32))[:N]


# speedup vs baseline: 26.0609x; 2.1977x over previous
import functools

import jax
import jax.numpy as jnp
from jax import lax
from jax.experimental import pallas as pl
from jax.experimental.pallas import tpu as pltpu
from jax.experimental.pallas import tpu_sc as plsc

N = 50000
E = 800000
NC = 2
NS = 16
NP = 51200
NA = 50176
RPT = NA // NS
CH = 50
TROWS = E // CH // NS
G = 40
NBUF = 10
CHD = 25
DTROWS = E // CHD // (NC * NS)
BN = 1024
NBLK = NP // BN

_mesh = functools.partial(
    plsc.VectorSubcoreMesh,
    core_axis_name="c", subcore_axis_name="s",
    num_cores=NC, num_subcores=NS,
)


@functools.partial(
    pl.kernel,
    out_type=jax.ShapeDtypeStruct((NC * NP,), jnp.float32),
    mesh=_mesh(),
    compiler_params=pltpu.CompilerParams(use_tc_tiling_on_sc=False),
    scratch_types=[
        pltpu.VMEM_SHARED((NP,), jnp.float32),
        pltpu.VMEM((DTROWS, CHD), jnp.int32),
        pltpu.VMEM((CHD,), jnp.float32),
    ],
)
def _deg_kernel(dst_hbm, zeros_hbm, ones_hbm, out_hbm, acc, dstb, onesb):
    c = lax.axis_index("c")
    s = lax.axis_index("s")
    seg = pl.ds(s * (NP // NS), NP // NS)
    pltpu.sync_copy(zeros_hbm.at[seg], acc.at[seg])
    pltpu.sync_copy(ones_hbm, onesb)
    rowbase = (c * NS + s) * DTROWS
    pltpu.sync_copy(dst_hbm.at[pl.ds(rowbase, DTROWS)], dstb)
    plsc.subcore_barrier()

    @pl.loop(0, DTROWS)
    def _(j):
        pltpu.sync_copy(onesb, acc.at[dstb.at[j]], add=True)

    plsc.subcore_barrier()
    pltpu.sync_copy(acc.at[seg], out_hbm.at[pl.ds(c * NP + s * (NP // NS), NP // NS)])


def _make_edge_pass(F):
    @functools.partial(
        pl.kernel,
        out_type=jax.ShapeDtypeStruct((2 * NP, F), jnp.float32),
        mesh=_mesh(),
        compiler_params=pltpu.CompilerParams(use_tc_tiling_on_sc=False),
        scratch_types=[
            pltpu.VMEM_SHARED((NA, F), jnp.float32),
            pltpu.VMEM((G, CH), jnp.int32),
            pltpu.VMEM((G, CH), jnp.int32),
            [pltpu.VMEM((CH, F), jnp.float32) for _ in range(NBUF)],
            pltpu.SemaphoreType.DMA((NBUF,)),
            pltpu.SemaphoreType.DMA((NBUF,)),
        ],
    )
    def edge_pass(y_hbm, src_hbm, dst_hbm, out_hbm, acc, srcb, dstb, rows,
                  gsem, ssem):
        c = lax.axis_index("c")
        s = lax.axis_index("s")
        base = s * RPT
        pltpu.sync_copy(y_hbm.at[pl.ds(c * NP + base, RPT)], acc.at[pl.ds(base, RPT)])
        plsc.subcore_barrier()

        @pl.loop(0, TROWS // G)
        def _(g):
            pltpu.sync_copy(
                src_hbm.at[pl.ds((c * NS + s) * TROWS + g * G, G)], srcb)
            pltpu.sync_copy(dst_hbm.at[pl.ds(s * TROWS + g * G, G)], dstb)

            @pl.loop(0, G // NBUF)
            def _(t):
                gets = [
                    pltpu.async_copy(
                        y_hbm.at[srcb.at[t * NBUF + b]], rows[b], gsem.at[b])
                    for b in range(NBUF)
                ]
                puts = []
                for b in range(NBUF):
                    gets[b].wait()
                    puts.append(pltpu.async_copy(
                        rows[b], acc.at[dstb.at[t * NBUF + b]], ssem.at[b],
                        add=True))
                for b in range(NBUF):
                    puts[b].wait()

        plsc.subcore_barrier()
        pltpu.sync_copy(acc.at[pl.ds(base, RPT)], out_hbm.at[pl.ds(c * NP + base, RPT)])

    return edge_pass


_edge_pass_32 = _make_edge_pass(32)
_edge_pass_16 = _make_edge_pass(16)


def _dinv_block(degp_ref):
    return lax.rsqrt(degp_ref[0, 0] + degp_ref[0, 1] + 1.0)


def _y1_body(emb_ref, w_ref, degp_ref, out_ref):
    dinv = _dinv_block(degp_ref)
    xw = jnp.dot(emb_ref[...], w_ref[...], preferred_element_type=jnp.float32)
    y = xw * dinv[:, None]
    out_ref[0] = y[:, :32]
    out_ref[1] = y[:, 32:]


def _tc_y1(emb, W1, degp):
    return pl.pallas_call(
        _y1_body,
        grid=(NBLK,),
        in_specs=[
            pl.BlockSpec((BN, 64), lambda i: (i, 0)),
            pl.BlockSpec((64, 64), lambda i: (0, 0)),
            pl.BlockSpec((1, 2, BN), lambda i: (i, 0, 0)),
        ],
        out_specs=pl.BlockSpec((2, BN, 32), lambda i: (0, i, 0)),
        out_shape=jax.ShapeDtypeStruct((2, NP, 32), jnp.float32),
    )(emb, W1, degp)


def _y2_body(acc_ref, degp_ref, b1_ref, w2_ref, out_ref):
    dinv = _dinv_block(degp_ref)
    a = jnp.concatenate([acc_ref[0], acc_ref[1]], axis=1)
    h = jnp.maximum(a * dinv[:, None] + b1_ref[...], 0.0)
    xw2 = jnp.dot(h, w2_ref[...], preferred_element_type=jnp.float32)
    y2 = xw2 * dinv[:, None]
    out_ref[0] = y2[:, :16]
    out_ref[1] = y2[:, 16:]


def _tc_y2(acc1, degp, b1, W2):
    return pl.pallas_call(
        _y2_body,
        grid=(NBLK,),
        in_specs=[
            pl.BlockSpec((2, BN, 32), lambda i: (0, i, 0)),
            pl.BlockSpec((1, 2, BN), lambda i: (i, 0, 0)),
            pl.BlockSpec((1, 64), lambda i: (0, 0)),
            pl.BlockSpec((64, 32), lambda i: (0, 0)),
        ],
        out_specs=pl.BlockSpec((2, BN, 16), lambda i: (0, i, 0)),
        out_shape=jax.ShapeDtypeStruct((2, NP, 16), jnp.float32),
    )(acc1, degp, b1, W2)


def _final_body(acc_ref, degp_ref, b2_ref, out_ref):
    dinv = _dinv_block(degp_ref)
    a = jnp.concatenate([acc_ref[0], acc_ref[1]], axis=1)
    out_ref[...] = a * dinv[:, None] + b2_ref[...]


def _tc_final(acc2, degp, b2):
    return pl.pallas_call(
        _final_body,
        grid=(NBLK,),
        in_specs=[
            pl.BlockSpec((2, BN, 16), lambda i: (0, i, 0)),
            pl.BlockSpec((1, 2, BN), lambda i: (i, 0, 0)),
            pl.BlockSpec((1, 32), lambda i: (0, 0)),
        ],
        out_specs=pl.BlockSpec((BN, 32), lambda i: (i, 0)),
        out_shape=jax.ShapeDtypeStruct((NP, 32), jnp.float32),
    )(acc2, degp, b2)


@jax.jit
def kernel(edge_index, emb, W1, b1, W2, b2):
    src = edge_index[0].astype(jnp.int32)
    dst = edge_index[1].astype(jnp.int32)
    src2d = src.reshape(E // CH, CH)
    src_cat = jnp.concatenate([src2d, src2d + NP], axis=0)
    dst2d = dst.reshape(E // CH, CH)
    dstd = dst.reshape(E // CHD, CHD)

    zeros_np = jnp.zeros((NP,), jnp.float32)
    ones_c = jnp.ones((CHD,), jnp.float32)
    emb_p = jnp.pad(emb, ((0, NP - N), (0, 0)))

    degp = _deg_kernel(dstd, zeros_np, ones_c)
    degp3 = degp.reshape(2, NBLK, BN).transpose(1, 0, 2)
    y1 = _tc_y1(emb_p, W1, degp3)
    acc1 = _edge_pass_32(y1.reshape(2 * NP, 32), src_cat, dst2d)
    y2 = _tc_y2(acc1.reshape(2, NP, 32), degp3, b1.reshape(1, 64), W2)
    acc2 = _edge_pass_16(y2.reshape(2 * NP, 16), src_cat, dst2d)
    return _tc_final(acc2.reshape(2, NP, 16), degp3, b2.reshape(1, 32))[:N]


# speedup vs baseline: 29.1121x; 1.1171x over previous
import functools

import jax
import jax.numpy as jnp
from jax import lax
from jax.experimental import pallas as pl
from jax.experimental.pallas import tpu as pltpu
from jax.experimental.pallas import tpu_sc as plsc

N = 50000
E = 800000
NC = 2
NS = 16
NP = 51200
NA = 50176
RPT = NA // NS
CH = 50
EROWS = E // CH
TROWS = EROWS // NS
DROWS = EROWS // (NC * NS)
G = 40
NBUF = 10
BN = 1024
NBLK = NP // BN

_mesh = functools.partial(
    plsc.VectorSubcoreMesh,
    core_axis_name="c", subcore_axis_name="s",
    num_cores=NC, num_subcores=NS,
)
_sc_params = pltpu.CompilerParams(use_tc_tiling_on_sc=False)


@functools.partial(
    pl.kernel,
    out_type=jax.ShapeDtypeStruct((NC * NP,), jnp.float32),
    mesh=_mesh(),
    compiler_params=_sc_params,
    scratch_types=[
        pltpu.VMEM_SHARED((NP,), jnp.float32),
        pltpu.VMEM((DROWS, CH), jnp.int32),
        pltpu.VMEM((CH,), jnp.float32),
        pltpu.SemaphoreType.DMA((NBUF,)),
    ],
)
def _deg_kernel(e3_hbm, zeros_hbm, ones_hbm, out_hbm, acc, dstb, onesb, ssem):
    c = lax.axis_index("c")
    s = lax.axis_index("s")
    seg = pl.ds(s * (NP // NS), NP // NS)
    pltpu.sync_copy(zeros_hbm.at[seg], acc.at[seg])
    pltpu.sync_copy(ones_hbm, onesb)
    pltpu.sync_copy(e3_hbm.at[1, pl.ds((c * NS + s) * DROWS, DROWS)], dstb)
    plsc.subcore_barrier()

    @pl.loop(0, DROWS // NBUF)
    def _(t):
        puts = [
            pltpu.async_copy(
                onesb, acc.at[dstb.at[t * NBUF + b]], ssem.at[b], add=True)
            for b in range(NBUF)
        ]
        for p in puts:
            p.wait()

    plsc.subcore_barrier()
    pltpu.sync_copy(acc.at[seg], out_hbm.at[pl.ds(c * NP + s * (NP // NS), NP // NS)])


def _make_edge_pass(F):
    @functools.partial(
        pl.kernel,
        out_type=[
            jax.ShapeDtypeStruct((NP, F), jnp.float32),
            jax.ShapeDtypeStruct((NP, F), jnp.float32),
        ],
        mesh=_mesh(),
        compiler_params=_sc_params,
        scratch_types=[
            pltpu.VMEM_SHARED((NA, F), jnp.float32),
            pltpu.VMEM((G, CH), jnp.int32),
            pltpu.VMEM((G, CH), jnp.int32),
            [pltpu.VMEM((CH, F), jnp.float32) for _ in range(NBUF)],
            pltpu.SemaphoreType.DMA((NBUF,)),
            pltpu.SemaphoreType.DMA((NBUF,)),
        ],
    )
    def edge_pass(ya_hbm, yb_hbm, e3_hbm, outa_hbm, outb_hbm,
                  acc, srcb, dstb, rows, gsem, ssem):
        c = lax.axis_index("c")
        s = lax.axis_index("s")
        base = s * RPT

        def run(y_hbm, out_hbm):
            pltpu.sync_copy(y_hbm.at[pl.ds(base, RPT)], acc.at[pl.ds(base, RPT)])
            plsc.subcore_barrier()

            @pl.loop(0, TROWS // G)
            def _(g):
                row0 = s * TROWS + g * G
                pltpu.sync_copy(e3_hbm.at[0, pl.ds(row0, G)], srcb)
                pltpu.sync_copy(e3_hbm.at[1, pl.ds(row0, G)], dstb)

                @pl.loop(0, G // NBUF)
                def _(t):
                    gets = [
                        pltpu.async_copy(
                            y_hbm.at[srcb.at[t * NBUF + b]], rows[b],
                            gsem.at[b])
                        for b in range(NBUF)
                    ]
                    puts = []
                    for b in range(NBUF):
                        gets[b].wait()
                        puts.append(pltpu.async_copy(
                            rows[b], acc.at[dstb.at[t * NBUF + b]],
                            ssem.at[b], add=True))
                    for p in puts:
                        p.wait()

            plsc.subcore_barrier()
            pltpu.sync_copy(acc.at[pl.ds(base, RPT)], out_hbm.at[pl.ds(base, RPT)])

        @pl.when(c == 0)
        def _():
            run(ya_hbm, outa_hbm)

        @pl.when(c == 1)
        def _():
            run(yb_hbm, outb_hbm)

    return edge_pass


_edge_pass_32 = _make_edge_pass(32)
_edge_pass_16 = _make_edge_pass(16)


def _dinv_block(d0_ref, d1_ref):
    return lax.rsqrt(d0_ref[...] + d1_ref[...] + 1.0)


def _deg_specs():
    return [
        pl.BlockSpec((BN,), lambda i: (i,)),
        pl.BlockSpec((BN,), lambda i: (NBLK + i,)),
    ]


def _y1_body(emb_ref, w_ref, d0_ref, d1_ref, outa_ref, outb_ref):
    dinv = _dinv_block(d0_ref, d1_ref)
    xw = jnp.dot(emb_ref[...], w_ref[...], preferred_element_type=jnp.float32)
    y = xw * dinv[:, None]
    outa_ref[...] = y[:, :32]
    outb_ref[...] = y[:, 32:]


def _tc_y1(emb, W1, degp):
    return pl.pallas_call(
        _y1_body,
        grid=(NBLK,),
        in_specs=[
            pl.BlockSpec((BN, 64), lambda i: (i, 0)),
            pl.BlockSpec((64, 64), lambda i: (0, 0)),
        ] + _deg_specs(),
        out_specs=[
            pl.BlockSpec((BN, 32), lambda i: (i, 0)),
            pl.BlockSpec((BN, 32), lambda i: (i, 0)),
        ],
        out_shape=[
            jax.ShapeDtypeStruct((NP, 32), jnp.float32),
            jax.ShapeDtypeStruct((NP, 32), jnp.float32),
        ],
    )(emb, W1, degp, degp)


def _y2_body(acca_ref, accb_ref, d0_ref, d1_ref, b1_ref, w2_ref,
             outa_ref, outb_ref):
    dinv = _dinv_block(d0_ref, d1_ref)
    a = jnp.concatenate([acca_ref[...], accb_ref[...]], axis=1)
    h = jnp.maximum(a * dinv[:, None] + b1_ref[...], 0.0)
    xw2 = jnp.dot(h, w2_ref[...], preferred_element_type=jnp.float32)
    y2 = xw2 * dinv[:, None]
    outa_ref[...] = y2[:, :16]
    outb_ref[...] = y2[:, 16:]


def _tc_y2(acca, accb, degp, b1, W2):
    return pl.pallas_call(
        _y2_body,
        grid=(NBLK,),
        in_specs=[
            pl.BlockSpec((BN, 32), lambda i: (i, 0)),
            pl.BlockSpec((BN, 32), lambda i: (i, 0)),
        ] + _deg_specs() + [
            pl.BlockSpec((1, 64), lambda i: (0, 0)),
            pl.BlockSpec((64, 32), lambda i: (0, 0)),
        ],
        out_specs=[
            pl.BlockSpec((BN, 16), lambda i: (i, 0)),
            pl.BlockSpec((BN, 16), lambda i: (i, 0)),
        ],
        out_shape=[
            jax.ShapeDtypeStruct((NP, 16), jnp.float32),
            jax.ShapeDtypeStruct((NP, 16), jnp.float32),
        ],
    )(acca, accb, degp, degp, b1, W2)


def _final_body(acca_ref, accb_ref, d0_ref, d1_ref, b2_ref, out_ref):
    dinv = _dinv_block(d0_ref, d1_ref)
    a = jnp.concatenate([acca_ref[...], accb_ref[...]], axis=1)
    out_ref[...] = a * dinv[:, None] + b2_ref[...]


def _tc_final(acca, accb, degp, b2):
    return pl.pallas_call(
        _final_body,
        grid=(NBLK,),
        in_specs=[
            pl.BlockSpec((BN, 16), lambda i: (i, 0)),
            pl.BlockSpec((BN, 16), lambda i: (i, 0)),
        ] + _deg_specs() + [
            pl.BlockSpec((1, 32), lambda i: (0, 0)),
        ],
        out_specs=pl.BlockSpec((BN, 32), lambda i: (i, 0)),
        out_shape=jax.ShapeDtypeStruct((NP, 32), jnp.float32),
    )(acca, accb, degp, degp, b2)


@jax.jit
def kernel(edge_index, emb, W1, b1, W2, b2):
    e3 = edge_index.astype(jnp.int32).reshape(2, EROWS, CH)
    zeros_np = jnp.zeros((NP,), jnp.float32)
    ones_c = jnp.ones((CH,), jnp.float32)
    emb_p = jnp.pad(emb, ((0, NP - N), (0, 0)))

    degp = _deg_kernel(e3, zeros_np, ones_c)
    ya1, yb1 = _tc_y1(emb_p, W1, degp)
    acca1, accb1 = _edge_pass_32(ya1, yb1, e3)
    y2a, y2b = _tc_y2(acca1, accb1, degp, b1.reshape(1, 64), W2)
    acca2, accb2 = _edge_pass_16(y2a, y2b, e3)
    return _tc_final(acca2, accb2, degp, b2.reshape(1, 32))[:N]
